# baseline (speedup 1.0000x reference)
import functools
import math

import jax
import jax.numpy as jnp
from jax import lax
from jax.experimental import pallas as pl
from jax.experimental.pallas import tpu as pltpu
from jax.experimental.pallas import tpu_sc as plsc

D = 512
N_TOK = 8192
SCALE = 1.0 / math.sqrt(D)

_info = plsc.get_sparse_core_info()
_NC, _NS, _L = _info.num_cores, _info.num_subcores, _info.num_lanes
_NW = _NC * _NS
_TPW = N_TOK // _NW
_CH = 64
_NCHUNK = _TPW // _CH


def _emb_body(ids_hbm, tid_hbm, gid_hbm, word_hbm, task_hbm, seg_hbm,
              out_hbm,
              idx_v, cidx_v, tmp_v, tt_v, st_v, comb_v,
              rows_v0, rows_v1, rows_v2,
              gsem0, gsem1, gsem2, osem0, osem1, osem2):
    cid = lax.axis_index("c")
    sid = lax.axis_index("s")
    base = (sid * _NC + cid) * _TPW
    rows = (rows_v0, rows_v1, rows_v2)
    gsem = (gsem0, gsem1, gsem2)
    osem = (osem0, osem1, osem2)

    pltpu.sync_copy(ids_hbm.at[pl.ds(base, _TPW)], idx_v)

    cpw = [None, None, None]
    cpo = [None, None, None]

    def start(c):
        b = c % 3
        if cpo[b] is not None:
            cpo[b].wait()
        cpw[b] = pltpu.async_copy(
            word_hbm.at[idx_v.at[pl.ds(c * _CH, _CH)]], rows[b], gsem[b])

    start(0)
    start(1)

    cp_t = pltpu.async_copy(task_hbm, tt_v, osem0)
    cp_s = pltpu.async_copy(seg_hbm, st_v, osem1)
    cp_i = pltpu.async_copy(
        tid_hbm.at[pl.ds(base, _TPW)], cidx_v.at[pl.ds(0, _TPW)], osem2)
    pltpu.sync_copy(gid_hbm.at[pl.ds(base, _TPW)], tmp_v)
    cp_t.wait()
    cp_s.wait()
    cp_i.wait()

    def build9(j, carry):
        sl = pl.ds(j * _L, _L)
        for t in range(3):
            for g in range(3):
                comb_v[t * 3 + g, sl] = tt_v[t, sl] + st_v[g, sl] * SCALE
        return carry

    lax.fori_loop(0, D // _L, build9, 0)

    for j in range(_TPW // _L):
        sl = pl.ds(j * _L, _L)
        cidx_v[sl] = cidx_v[sl] * 3 + tmp_v[sl]
    for c in range(_NCHUNK):
        b = c % 3
        if c + 2 < _NCHUNK:
            start(c + 2)
        cpw[b].wait()

        @plsc.parallel_loop(0, _CH, unroll=3)
        def tok(i, b=b, c=c):
            cc = cidx_v[pl.ds(c * _CH + i, _L)][0]
            for j in range(D // _L):
                sl = pl.ds(j * _L, _L)
                plsc.addupdate(rows[b].at[i, sl], comb_v[cc, sl])

        cpo[b] = pltpu.async_copy(
            rows[b], out_hbm.at[pl.ds(base + c * _CH, _CH)], osem[b])
    for b in range(3):
        cpo[b].wait()


_emb_kernel = functools.partial(
    pl.kernel,
    out_type=jax.ShapeDtypeStruct((N_TOK, D), jnp.float32),
    mesh=plsc.VectorSubcoreMesh(core_axis_name="c", subcore_axis_name="s"),
    scratch_types=[
        pltpu.VMEM((_TPW,), jnp.int32),
        pltpu.VMEM((_TPW + _L,), jnp.int32),
        pltpu.VMEM((_TPW,), jnp.int32),
        pltpu.VMEM((3, D), jnp.float32),
        pltpu.VMEM((3, D), jnp.float32),
        pltpu.VMEM((9, D), jnp.float32),
        pltpu.VMEM((_CH, D), jnp.float32),
        pltpu.VMEM((_CH, D), jnp.float32),
        pltpu.VMEM((_CH, D), jnp.float32),
        pltpu.SemaphoreType.DMA,
        pltpu.SemaphoreType.DMA,
        pltpu.SemaphoreType.DMA,
        pltpu.SemaphoreType.DMA,
        pltpu.SemaphoreType.DMA,
        pltpu.SemaphoreType.DMA,
    ],
)(_emb_body)


@jax.jit
def kernel(input_ids, task_ids, segment_ids, word_table, task_table,
           segment_table):
    shape = input_ids.shape
    ids = input_ids.reshape(-1).astype(jnp.int32)
    tid = task_ids.reshape(-1).astype(jnp.int32)
    gid = segment_ids.reshape(-1).astype(jnp.int32)
    out = _emb_kernel(ids, tid, gid, word_table, task_table, segment_table)
    return out.reshape(shape + (D,))


# speedup vs baseline: 1.0829x; 1.0829x over previous
import functools
import math

import jax
import jax.numpy as jnp
from jax import lax
from jax.experimental import pallas as pl
from jax.experimental.pallas import tpu as pltpu
from jax.experimental.pallas import tpu_sc as plsc

D = 512
N_TOK = 8192
SCALE = 1.0 / math.sqrt(D)

_info = plsc.get_sparse_core_info()
_NC, _NS, _L = _info.num_cores, _info.num_subcores, _info.num_lanes
_NW = _NC * _NS
_TPW = N_TOK // _NW
_CH = 64
_NCHUNK = _TPW // _CH


def _emb_body(ids_hbm, tid_hbm, gid_hbm, word_hbm, task_hbm, seg_hbm,
              out_hbm,
              idx_v, cidx_v, tmp_v, tt_v, st_v, comb_v,
              rows_v0, rows_v1, rows_v2,
              gsem0, gsem1, gsem2, osem0, osem1, osem2):
    cid = lax.axis_index("c")
    sid = lax.axis_index("s")
    base = (sid * _NC + cid) * _TPW
    rows = (rows_v0, rows_v1, rows_v2)
    gsem = (gsem0, gsem1, gsem2)
    osem = (osem0, osem1, osem2)

    pltpu.sync_copy(ids_hbm.at[pl.ds(base, _TPW)], idx_v)

    cpw = [None, None, None]
    cpo = [None, None, None]

    def start(c):
        b = c % 3
        if cpo[b] is not None:
            cpo[b].wait()
        cpw[b] = pltpu.async_copy(
            word_hbm.at[idx_v.at[pl.ds(c * _CH, _CH)]], rows[b], gsem[b])

    start(0)
    start(1)

    cp_t = pltpu.async_copy(task_hbm, tt_v, osem0)
    cp_s = pltpu.async_copy(seg_hbm, st_v, osem1)
    cp_i = pltpu.async_copy(
        tid_hbm.at[pl.ds(base, _TPW)], cidx_v.at[pl.ds(0, _TPW)], osem2)
    pltpu.sync_copy(gid_hbm.at[pl.ds(base, _TPW)], tmp_v)
    cp_t.wait()
    cp_s.wait()
    cp_i.wait()

    def build9(j, carry):
        sl = pl.ds(j * _L, _L)
        for t in range(3):
            for g in range(3):
                comb_v[t * 3 + g, sl] = tt_v[t, sl] + st_v[g, sl] * SCALE
        return carry

    lax.fori_loop(0, D // _L, build9, 0)

    for j in range(_TPW // _L):
        sl = pl.ds(j * _L, _L)
        cidx_v[sl] = cidx_v[sl] * 3 + tmp_v[sl]
    for c in range(_NCHUNK):
        b = c % 3
        if c + 2 < _NCHUNK:
            start(c + 2)
        cpw[b].wait()

        @plsc.parallel_loop(0, _CH, unroll=2)
        def tok(i, b=b, c=c):
            cc = cidx_v[pl.ds(c * _CH + i, _L)][0]
            for j in range(D // _L):
                sl = pl.ds(j * _L, _L)
                plsc.addupdate(rows[b].at[i, sl], comb_v[cc, sl])

        cpo[b] = pltpu.async_copy(
            rows[b], out_hbm.at[pl.ds(base + c * _CH, _CH)], osem[b])
    for b in range(3):
        cpo[b].wait()


_emb_kernel = functools.partial(
    pl.kernel,
    out_type=jax.ShapeDtypeStruct((N_TOK, D), jnp.float32),
    mesh=plsc.VectorSubcoreMesh(core_axis_name="c", subcore_axis_name="s"),
    scratch_types=[
        pltpu.VMEM((_TPW,), jnp.int32),
        pltpu.VMEM((_TPW + _L,), jnp.int32),
        pltpu.VMEM((_TPW,), jnp.int32),
        pltpu.VMEM((3, D), jnp.float32),
        pltpu.VMEM((3, D), jnp.float32),
        pltpu.VMEM((9, D), jnp.float32),
        pltpu.VMEM((_CH, D), jnp.float32),
        pltpu.VMEM((_CH, D), jnp.float32),
        pltpu.VMEM((_CH, D), jnp.float32),
        pltpu.SemaphoreType.DMA,
        pltpu.SemaphoreType.DMA,
        pltpu.SemaphoreType.DMA,
        pltpu.SemaphoreType.DMA,
        pltpu.SemaphoreType.DMA,
        pltpu.SemaphoreType.DMA,
    ],
)(_emb_body)


@jax.jit
def kernel(input_ids, task_ids, segment_ids, word_table, task_table,
           segment_table):
    shape = input_ids.shape
    ids = input_ids.reshape(-1).astype(jnp.int32)
    tid = task_ids.reshape(-1).astype(jnp.int32)
    gid = segment_ids.reshape(-1).astype(jnp.int32)
    out = _emb_kernel(ids, tid, gid, word_table, task_table, segment_table)
    return out.reshape(shape + (D,))


# speedup vs baseline: 1.0969x; 1.0129x over previous
import functools
import math

import jax
import jax.numpy as jnp
from jax import lax
from jax.experimental import pallas as pl
from jax.experimental.pallas import tpu as pltpu
from jax.experimental.pallas import tpu_sc as plsc

D = 512
N_TOK = 8192
SCALE = 1.0 / math.sqrt(D)

_info = plsc.get_sparse_core_info()
_NC, _NS, _L = _info.num_cores, _info.num_subcores, _info.num_lanes
_NW = _NC * _NS
_TPW = N_TOK // _NW
_CH = 64
_NCHUNK = _TPW // _CH


def _emb_body(ids_hbm, tid_hbm, gid_hbm, word_hbm, task_hbm, seg_hbm,
              out_hbm,
              idx_v, cidx_v, tmp_v, tt_v, st_v, comb_v,
              rows_v0, rows_v1, rows_v2,
              gsem0, gsem1, gsem2, osem0, osem1, osem2):
    cid = lax.axis_index("c")
    sid = lax.axis_index("s")
    base = (sid * _NC + cid) * _TPW
    rows = (rows_v0, rows_v1, rows_v2)
    gsem = (gsem0, gsem1, gsem2)
    osem = (osem0, osem1, osem2)

    pltpu.sync_copy(ids_hbm.at[pl.ds(base, _TPW)], idx_v)

    cpw = [None, None, None]
    cpo = [None, None, None]

    def start(c):
        b = c % 3
        if cpo[b] is not None:
            cpo[b].wait()
        h = _CH // 2
        cpw[b] = (
            pltpu.async_copy(
                word_hbm.at[idx_v.at[pl.ds(c * _CH, h)]],
                rows[b].at[pl.ds(0, h)], gsem[b]),
            pltpu.async_copy(
                word_hbm.at[idx_v.at[pl.ds(c * _CH + h, h)]],
                rows[b].at[pl.ds(h, h)], osem[b]))

    start(0)
    start(1)

    cp_t = pltpu.async_copy(task_hbm, tt_v, osem0)
    cp_s = pltpu.async_copy(seg_hbm, st_v, osem1)
    cp_i = pltpu.async_copy(
        tid_hbm.at[pl.ds(base, _TPW)], cidx_v.at[pl.ds(0, _TPW)], osem2)
    pltpu.sync_copy(gid_hbm.at[pl.ds(base, _TPW)], tmp_v)
    cp_t.wait()
    cp_s.wait()
    cp_i.wait()

    def build9(j, carry):
        sl = pl.ds(j * _L, _L)
        for t in range(3):
            for g in range(3):
                comb_v[t * 3 + g, sl] = tt_v[t, sl] + st_v[g, sl] * SCALE
        return carry

    lax.fori_loop(0, D // _L, build9, 0)

    for j in range(_TPW // _L):
        sl = pl.ds(j * _L, _L)
        cidx_v[sl] = cidx_v[sl] * 3 + tmp_v[sl]
    for c in range(_NCHUNK):
        b = c % 3
        if c + 2 < _NCHUNK:
            start(c + 2)
        cpw[b][0].wait()
        cpw[b][1].wait()

        @plsc.parallel_loop(0, _CH, unroll=2)
        def tok(i, b=b, c=c):
            cc = cidx_v[pl.ds(c * _CH + i, _L)][0]
            for j in range(D // _L):
                sl = pl.ds(j * _L, _L)
                plsc.addupdate(rows[b].at[i, sl], comb_v[cc, sl])

        cpo[b] = pltpu.async_copy(
            rows[b], out_hbm.at[pl.ds(base + c * _CH, _CH)], osem[b])
    for b in range(3):
        cpo[b].wait()


_emb_kernel = functools.partial(
    pl.kernel,
    out_type=jax.ShapeDtypeStruct((N_TOK, D), jnp.float32),
    mesh=plsc.VectorSubcoreMesh(core_axis_name="c", subcore_axis_name="s"),
    scratch_types=[
        pltpu.VMEM((_TPW,), jnp.int32),
        pltpu.VMEM((_TPW + _L,), jnp.int32),
        pltpu.VMEM((_TPW,), jnp.int32),
        pltpu.VMEM((3, D), jnp.float32),
        pltpu.VMEM((3, D), jnp.float32),
        pltpu.VMEM((9, D), jnp.float32),
        pltpu.VMEM((_CH, D), jnp.float32),
        pltpu.VMEM((_CH, D), jnp.float32),
        pltpu.VMEM((_CH, D), jnp.float32),
        pltpu.SemaphoreType.DMA,
        pltpu.SemaphoreType.DMA,
        pltpu.SemaphoreType.DMA,
        pltpu.SemaphoreType.DMA,
        pltpu.SemaphoreType.DMA,
        pltpu.SemaphoreType.DMA,
    ],
)(_emb_body)


@jax.jit
def kernel(input_ids, task_ids, segment_ids, word_table, task_table,
           segment_table):
    shape = input_ids.shape
    ids = input_ids.reshape(-1).astype(jnp.int32)
    tid = task_ids.reshape(-1).astype(jnp.int32)
    gid = segment_ids.reshape(-1).astype(jnp.int32)
    out = _emb_kernel(ids, tid, gid, word_table, task_table, segment_table)
    return out.reshape(shape + (D,))
